# baseline (speedup 1.0000x reference)
import functools

import jax
import jax.numpy as jnp
from jax import lax
from jax.experimental import pallas as pl
from jax.experimental.pallas import tpu as pltpu
from jax.experimental.pallas import tpu_sc as plsc

N_EMBD = 768
HALF = N_EMBD // 2
D_FF = 768
E = 64
CAP = 128
PRIME = 5099
B, T = 4, 2048
N = B * T
TRASH = E * CAP
EPG = 2
BUF_ROWS = (E + EPG) * CAP
NW = 32
TOK_PER_W = N // NW
T_BLK = 512
CHUNK_D = 128
NCHUNK_D = TOK_PER_W // CHUNK_D
CHUNK_C = 128
NCHUNK_C = TOK_PER_W // CHUNK_C


def _pack_rows(a):
    u = jax.lax.bitcast_convert_type(a, jnp.uint32)
    rnd = jnp.uint32(0x8000)
    lo = (u[:, :HALF] + rnd) >> 16
    hi = (u[:, HALF:] + rnd) & jnp.uint32(0xFFFF0000)
    return jax.lax.bitcast_convert_type(lo | hi, jnp.int32)


def _unpack_rows(w):
    u = jax.lax.bitcast_convert_type(w, jnp.uint32)
    left = jax.lax.bitcast_convert_type(u << 16, jnp.float32)
    right = jax.lax.bitcast_convert_type(u & jnp.uint32(0xFFFF0000), jnp.float32)
    return jnp.concatenate([left, right], axis=1)


def _mix_route_body(x_ref, ss_ref, maak_ref, maar_ref, wr_ref, tid_ref,
                    xk_ref, r_ref, loc_ref, cnt_ref, prevrow_ref):
    b = pl.program_id(0)
    th = pl.program_id(1)

    @pl.when((b == 0) & (th == 0))
    def _init():
        cnt_ref[...] = jnp.zeros((E, 1), jnp.int32)

    xv = x_ref[0]
    prev = jnp.where(th == 0, ss_ref[0], prevrow_ref[...])
    prevrow_ref[...] = xv[T_BLK - 1:T_BLK]
    xprev = jnp.concatenate([prev, xv[:-1]], axis=0)
    dx = xprev - xv
    xk_ref[0] = _pack_rows(xv + dx * maak_ref[...])
    xr = xv + dx * maar_ref[...]
    r_ref[0] = _pack_rows(jax.nn.sigmoid(
        lax.dot_general(xr.astype(jnp.bfloat16),
                        wr_ref[...].astype(jnp.bfloat16),
                        (((1,), (1,)), ((), ())),
                        preferred_element_type=jnp.float32)))

    tid = tid_ref[0]
    e = (tid * PRIME) % E
    erow = lax.broadcasted_iota(jnp.int32, (E, T_BLK), 0)
    oh = (erow == e).astype(jnp.int32)
    c = oh
    k = 1
    while k < T_BLK:
        c = c + jnp.concatenate(
            [jnp.zeros((E, k), jnp.int32), c[:, :-k]], axis=1)
        k *= 2
    ctot = c + cnt_ref[...]
    cnt_ref[...] = ctot[:, T_BLK - 1:T_BLK]
    pos = jnp.sum(ctot * oh, axis=0, keepdims=True) - 1
    keep = pos < CAP
    loc = e * CAP + jnp.minimum(pos, CAP - 1)
    loc_ref[...] = jnp.where(keep, loc, TRASH).reshape(1, 1, T_BLK)


def _mix_route(x, ss3, maak2, maar2, Wr, tid3):
    nt = T // T_BLK
    return pl.pallas_call(
        _mix_route_body,
        grid=(B, nt),
        in_specs=[
            pl.BlockSpec((1, T_BLK, N_EMBD), lambda b, t: (b, t, 0)),
            pl.BlockSpec((1, 1, N_EMBD), lambda b, t: (b, 0, 0)),
            pl.BlockSpec((1, N_EMBD), lambda b, t: (0, 0)),
            pl.BlockSpec((1, N_EMBD), lambda b, t: (0, 0)),
            pl.BlockSpec((N_EMBD, N_EMBD), lambda b, t: (0, 0)),
            pl.BlockSpec((1, 1, T_BLK), lambda b, t: (b, 0, t)),
        ],
        out_specs=[
            pl.BlockSpec((1, T_BLK, HALF), lambda b, t: (b, t, 0)),
            pl.BlockSpec((1, T_BLK, HALF), lambda b, t: (b, t, 0)),
            pl.BlockSpec((1, 1, T_BLK), lambda b, t: (b, 0, t)),
        ],
        out_shape=[
            jax.ShapeDtypeStruct((B, T, HALF), jnp.int32),
            jax.ShapeDtypeStruct((B, T, HALF), jnp.int32),
            jax.ShapeDtypeStruct((B, 1, T), jnp.int32),
        ],
        scratch_shapes=[pltpu.VMEM((E, 1), jnp.int32),
                        pltpu.VMEM((1, N_EMBD), jnp.float32)],
    )(x, ss3, maak2, maar2, Wr, tid3)


def _make_dispatch():
    mesh = plsc.VectorSubcoreMesh(core_axis_name="c", subcore_axis_name="s")

    @functools.partial(
        pl.kernel,
        mesh=mesh,
        out_type=jax.ShapeDtypeStruct((BUF_ROWS, HALF), jnp.int32),
        scratch_types=[
            pltpu.VMEM((NCHUNK_D, CHUNK_D), jnp.int32),
            pltpu.VMEM((CHUNK_D, HALF), jnp.int32),
            pltpu.VMEM((CHUNK_D, HALF), jnp.int32),
            pltpu.SemaphoreType.DMA,
            pltpu.SemaphoreType.DMA,
            pltpu.SemaphoreType.DMA,
            pltpu.SemaphoreType.DMA,
        ],
    )
    def disp(xk_hbm, loc_hbm, buf_hbm, idx_v, row0, row1,
             is0, is1, os0, os1):
        wid = lax.axis_index("s") * 2 + lax.axis_index("c")
        base = wid * TOK_PER_W
        pltpu.sync_copy(loc_hbm.at[pl.ds(wid * NCHUNK_D, NCHUNK_D)], idx_v)
        rows = (row0, row1)
        isems = (is0, is1)
        osems = (os0, os1)

        def cin(j):
            return pltpu.make_async_copy(
                xk_hbm.at[pl.ds(base + j * CHUNK_D, CHUNK_D)],
                rows[j % 2], isems[j % 2])

        def cout(j):
            return pltpu.make_async_copy(
                rows[j % 2], buf_hbm.at[idx_v.at[j]], osems[j % 2])

        cin(0).start()
        for j in range(NCHUNK_D):
            if j + 1 < NCHUNK_D:
                if j >= 1:
                    cout(j - 1).wait()
                cin(j + 1).start()
            cin(j).wait()
            cout(j).start()
        cout(NCHUNK_D - 2).wait()
        cout(NCHUNK_D - 1).wait()

    return disp


def _make_combine():
    mesh = plsc.VectorSubcoreMesh(core_axis_name="c", subcore_axis_name="s")

    @functools.partial(
        pl.kernel,
        mesh=mesh,
        out_type=jax.ShapeDtypeStruct((N, HALF), jnp.int32),
        scratch_types=[
            pltpu.VMEM((NCHUNK_C, CHUNK_C), jnp.int32),
            pltpu.VMEM((CHUNK_C, HALF), jnp.int32),
            pltpu.VMEM((CHUNK_C, HALF), jnp.int32),
            pltpu.SemaphoreType.DMA,
            pltpu.SemaphoreType.DMA,
            pltpu.SemaphoreType.DMA,
            pltpu.SemaphoreType.DMA,
        ],
    )
    def comb(outbuf_hbm, loc_hbm, y_hbm, idx_v, row0, row1,
             gs0, gs1, os0, os1):
        wid = lax.axis_index("s") * 2 + lax.axis_index("c")
        base = wid * TOK_PER_W
        pltpu.sync_copy(loc_hbm.at[pl.ds(wid * NCHUNK_C, NCHUNK_C)], idx_v)
        rows = (row0, row1)
        gsems = (gs0, gs1)
        osems = (os0, os1)

        def gin(j):
            return pltpu.make_async_copy(
                outbuf_hbm.at[idx_v.at[j]], rows[j % 2], gsems[j % 2])

        def oput(j):
            return pltpu.make_async_copy(
                rows[j % 2], y_hbm.at[pl.ds(base + j * CHUNK_C, CHUNK_C)],
                osems[j % 2])

        gin(0).start()
        for j in range(NCHUNK_C):
            if j + 1 < NCHUNK_C:
                if j >= 1:
                    oput(j - 1).wait()
                gin(j + 1).start()
            gin(j).wait()
            oput(j).start()
        oput(NCHUNK_C - 2).wait()
        oput(NCHUNK_C - 1).wait()

    return comb


def _ffn_body(buf_ref, wk_ref, wv_ref, out_ref):
    g = pl.program_id(0)
    ng = E // EPG

    @pl.when(g < ng)
    def _compute():
        for i in range(EPG):
            a = _unpack_rows(buf_ref[pl.ds(i * CAP, CAP), :])
            h = jnp.dot(a, wk_ref[i], preferred_element_type=jnp.float32)
            h = jnp.square(jnp.maximum(h, 0.0))
            out_ref[pl.ds(i * CAP, CAP), :] = _pack_rows(jnp.dot(
                h, wv_ref[i], preferred_element_type=jnp.float32))

    @pl.when(g == ng)
    def _zero():
        out_ref[...] = jnp.zeros((EPG * CAP, HALF), jnp.int32)


def _ffn(buf, Wk, Wv):
    ng = E // EPG
    return pl.pallas_call(
        _ffn_body,
        grid=(ng + 1,),
        in_specs=[
            pl.BlockSpec((EPG * CAP, HALF),
                         lambda g: (jnp.minimum(g, E // EPG - 1), 0)),
            pl.BlockSpec((EPG, N_EMBD, D_FF),
                         lambda g: (jnp.minimum(g, E // EPG - 1), 0, 0)),
            pl.BlockSpec((EPG, D_FF, N_EMBD),
                         lambda g: (jnp.minimum(g, E // EPG - 1), 0, 0)),
        ],
        out_specs=pl.BlockSpec((EPG * CAP, HALF), lambda g: (g, 0)),
        out_shape=jax.ShapeDtypeStruct((BUF_ROWS, HALF), jnp.int32),
    )(buf, Wk, Wv)


def _gate_body(r_ref, y_ref, out_ref):
    out_ref[0] = _unpack_rows(r_ref[0]) * _unpack_rows(y_ref[0])


def _gate(r, y3):
    return pl.pallas_call(
        _gate_body,
        grid=(B,),
        in_specs=[
            pl.BlockSpec((1, T, HALF), lambda b: (b, 0, 0)),
            pl.BlockSpec((1, T, HALF), lambda b: (b, 0, 0)),
        ],
        out_specs=pl.BlockSpec((1, T, N_EMBD), lambda b: (b, 0, 0)),
        out_shape=jax.ShapeDtypeStruct((B, T, N_EMBD), jnp.float32),
    )(r, y3)


def kernel(x, shift_state, time_maa_k, time_maa_r, Wk, Wv, Wr, token_ids):
    maak2 = time_maa_k.reshape(1, N_EMBD)
    maar2 = time_maa_r.reshape(1, N_EMBD)
    ss3 = shift_state.reshape(B, 1, N_EMBD)
    tid3 = token_ids.reshape(B, 1, T)

    xk, r, loc = _mix_route(x, ss3, maak2, maar2, Wr, tid3)

    loc2 = loc.reshape(N // CHUNK_D, CHUNK_D)
    buf = _make_dispatch()(xk.reshape(N, HALF), loc2)
    outbuf = _ffn(buf, Wk, Wv)
    y = _make_combine()(outbuf, loc2)
    out = _gate(r, y.reshape(B, T, HALF))
    return out, x[:, -1]


# speedup vs baseline: 1.0215x; 1.0215x over previous
import functools

import jax
import jax.numpy as jnp
from jax import lax
from jax.experimental import pallas as pl
from jax.experimental.pallas import tpu as pltpu
from jax.experimental.pallas import tpu_sc as plsc

N_EMBD = 768
HALF = N_EMBD // 2
D_FF = 768
E = 64
CAP = 128
PRIME = 5099
B, T = 4, 2048
N = B * T
TRASH = E * CAP
EPG = 2
BUF_ROWS = (E + EPG) * CAP
NW = 32
TOK_PER_W = N // NW
T_BLK = 1024
CHUNK_D = 128
NCHUNK_D = TOK_PER_W // CHUNK_D
CHUNK_C = 128
NCHUNK_C = TOK_PER_W // CHUNK_C


def _pack_rows(a):
    u = jax.lax.bitcast_convert_type(a, jnp.uint32)
    rnd = jnp.uint32(0x8000)
    lo = (u[:, :HALF] + rnd) >> 16
    hi = (u[:, HALF:] + rnd) & jnp.uint32(0xFFFF0000)
    return jax.lax.bitcast_convert_type(lo | hi, jnp.int32)


def _unpack_rows(w):
    u = jax.lax.bitcast_convert_type(w, jnp.uint32)
    left = jax.lax.bitcast_convert_type(u << 16, jnp.float32)
    right = jax.lax.bitcast_convert_type(u & jnp.uint32(0xFFFF0000), jnp.float32)
    return jnp.concatenate([left, right], axis=1)


def _mix_route_body(x_ref, ss_ref, maak_ref, maar_ref, wr_ref, tid_ref,
                    xk_ref, r_ref, loc_ref, cnt_ref, prevrow_ref):
    b = pl.program_id(0)
    th = pl.program_id(1)

    @pl.when((b == 0) & (th == 0))
    def _init():
        cnt_ref[...] = jnp.zeros((E, 1), jnp.int32)

    xv = x_ref[0]
    prev = jnp.where(th == 0, ss_ref[0], prevrow_ref[...])
    prevrow_ref[...] = xv[T_BLK - 1:T_BLK]
    xprev = jnp.concatenate([prev, xv[:-1]], axis=0)
    dx = xprev - xv
    xk_ref[...] = _pack_rows(xv + dx * maak_ref[...])
    xr = xv + dx * maar_ref[...]
    r_ref[...] = _pack_rows(jax.nn.sigmoid(
        lax.dot_general(xr.astype(jnp.bfloat16),
                        wr_ref[...].astype(jnp.bfloat16),
                        (((1,), (1,)), ((), ())),
                        preferred_element_type=jnp.float32)))

    tid = tid_ref[0]
    e = (tid * PRIME) % E
    erow = lax.broadcasted_iota(jnp.int32, (E, T_BLK), 0)
    oh = (erow == e).astype(jnp.int32)
    c = oh
    k = 1
    while k < T_BLK:
        c = c + jnp.concatenate(
            [jnp.zeros((E, k), jnp.int32), c[:, :-k]], axis=1)
        k *= 2
    ctot = c + cnt_ref[...]
    cnt_ref[...] = ctot[:, T_BLK - 1:T_BLK]
    pos = jnp.sum(ctot * oh, axis=0, keepdims=True) - 1
    keep = pos < CAP
    loc = e * CAP + jnp.minimum(pos, CAP - 1)
    loc_ref[...] = jnp.where(keep, loc, TRASH).reshape(T_BLK // CHUNK_D, CHUNK_D)


def _mix_route(x, ss3, maak2, maar2, Wr, tid3):
    nt = T // T_BLK
    return pl.pallas_call(
        _mix_route_body,
        grid=(B, nt),
        in_specs=[
            pl.BlockSpec((1, T_BLK, N_EMBD), lambda b, t: (b, t, 0)),
            pl.BlockSpec((1, 1, N_EMBD), lambda b, t: (b, 0, 0)),
            pl.BlockSpec((1, N_EMBD), lambda b, t: (0, 0)),
            pl.BlockSpec((1, N_EMBD), lambda b, t: (0, 0)),
            pl.BlockSpec((N_EMBD, N_EMBD), lambda b, t: (0, 0)),
            pl.BlockSpec((1, 1, T_BLK), lambda b, t: (b, 0, t)),
        ],
        out_specs=[
            pl.BlockSpec((T_BLK, HALF), lambda b, t: (b * (T // T_BLK) + t, 0)),
            pl.BlockSpec((T_BLK, HALF), lambda b, t: (b * (T // T_BLK) + t, 0)),
            pl.BlockSpec((T_BLK // CHUNK_D, CHUNK_D),
                         lambda b, t: (b * (T // T_BLK) + t, 0)),
        ],
        out_shape=[
            jax.ShapeDtypeStruct((N, HALF), jnp.int32),
            jax.ShapeDtypeStruct((N, HALF), jnp.int32),
            jax.ShapeDtypeStruct((N // CHUNK_D, CHUNK_D), jnp.int32),
        ],
        scratch_shapes=[pltpu.VMEM((E, 1), jnp.int32),
                        pltpu.VMEM((1, N_EMBD), jnp.float32)],
    )(x, ss3, maak2, maar2, Wr, tid3)


def _make_dispatch():
    mesh = plsc.VectorSubcoreMesh(core_axis_name="c", subcore_axis_name="s")

    @functools.partial(
        pl.kernel,
        mesh=mesh,
        out_type=jax.ShapeDtypeStruct((BUF_ROWS, HALF), jnp.int32),
        scratch_types=[
            pltpu.VMEM((NCHUNK_D, CHUNK_D), jnp.int32),
            pltpu.VMEM((CHUNK_D, HALF), jnp.int32),
            pltpu.VMEM((CHUNK_D, HALF), jnp.int32),
            pltpu.SemaphoreType.DMA,
            pltpu.SemaphoreType.DMA,
            pltpu.SemaphoreType.DMA,
            pltpu.SemaphoreType.DMA,
        ],
    )
    def disp(xk_hbm, loc_hbm, buf_hbm, idx_v, row0, row1,
             is0, is1, os0, os1):
        wid = lax.axis_index("s") * 2 + lax.axis_index("c")
        base = wid * TOK_PER_W
        pltpu.sync_copy(loc_hbm.at[pl.ds(wid * NCHUNK_D, NCHUNK_D)], idx_v)
        rows = (row0, row1)
        isems = (is0, is1)
        osems = (os0, os1)

        def cin(j):
            return pltpu.make_async_copy(
                xk_hbm.at[pl.ds(base + j * CHUNK_D, CHUNK_D)],
                rows[j % 2], isems[j % 2])

        def cout(j):
            return pltpu.make_async_copy(
                rows[j % 2], buf_hbm.at[idx_v.at[j]], osems[j % 2])

        cin(0).start()
        for j in range(NCHUNK_D):
            if j + 1 < NCHUNK_D:
                if j >= 1:
                    cout(j - 1).wait()
                cin(j + 1).start()
            cin(j).wait()
            cout(j).start()
        cout(NCHUNK_D - 2).wait()
        cout(NCHUNK_D - 1).wait()

    return disp


def _make_combine():
    mesh = plsc.VectorSubcoreMesh(core_axis_name="c", subcore_axis_name="s")

    @functools.partial(
        pl.kernel,
        mesh=mesh,
        out_type=jax.ShapeDtypeStruct((N, HALF), jnp.int32),
        scratch_types=[
            pltpu.VMEM((NCHUNK_C, CHUNK_C), jnp.int32),
            pltpu.VMEM((CHUNK_C, HALF), jnp.int32),
            pltpu.VMEM((CHUNK_C, HALF), jnp.int32),
            pltpu.SemaphoreType.DMA,
            pltpu.SemaphoreType.DMA,
            pltpu.SemaphoreType.DMA,
            pltpu.SemaphoreType.DMA,
        ],
    )
    def comb(outbuf_hbm, loc_hbm, y_hbm, idx_v, row0, row1,
             gs0, gs1, os0, os1):
        wid = lax.axis_index("s") * 2 + lax.axis_index("c")
        base = wid * TOK_PER_W
        pltpu.sync_copy(loc_hbm.at[pl.ds(wid * NCHUNK_C, NCHUNK_C)], idx_v)
        rows = (row0, row1)
        gsems = (gs0, gs1)
        osems = (os0, os1)

        def gin(j):
            return pltpu.make_async_copy(
                outbuf_hbm.at[idx_v.at[j]], rows[j % 2], gsems[j % 2])

        def oput(j):
            return pltpu.make_async_copy(
                rows[j % 2], y_hbm.at[pl.ds(base + j * CHUNK_C, CHUNK_C)],
                osems[j % 2])

        gin(0).start()
        for j in range(NCHUNK_C):
            if j + 1 < NCHUNK_C:
                if j >= 1:
                    oput(j - 1).wait()
                gin(j + 1).start()
            gin(j).wait()
            oput(j).start()
        oput(NCHUNK_C - 2).wait()
        oput(NCHUNK_C - 1).wait()

    return comb


def _ffn_body(buf_ref, wk_ref, wv_ref, out_ref):
    g = pl.program_id(0)
    ng = E // EPG

    @pl.when(g < ng)
    def _compute():
        for i in range(EPG):
            a = _unpack_rows(buf_ref[pl.ds(i * CAP, CAP), :])
            h = jnp.dot(a, wk_ref[i], preferred_element_type=jnp.float32)
            h = jnp.square(jnp.maximum(h, 0.0))
            out_ref[pl.ds(i * CAP, CAP), :] = _pack_rows(jnp.dot(
                h, wv_ref[i], preferred_element_type=jnp.float32))

    @pl.when(g == ng)
    def _zero():
        out_ref[...] = jnp.zeros((EPG * CAP, HALF), jnp.int32)


def _ffn(buf, Wk, Wv):
    ng = E // EPG
    return pl.pallas_call(
        _ffn_body,
        grid=(ng + 1,),
        in_specs=[
            pl.BlockSpec((EPG * CAP, HALF),
                         lambda g: (jnp.minimum(g, E // EPG - 1), 0)),
            pl.BlockSpec((EPG, N_EMBD, D_FF),
                         lambda g: (jnp.minimum(g, E // EPG - 1), 0, 0)),
            pl.BlockSpec((EPG, D_FF, N_EMBD),
                         lambda g: (jnp.minimum(g, E // EPG - 1), 0, 0)),
        ],
        out_specs=pl.BlockSpec((EPG * CAP, HALF), lambda g: (g, 0)),
        out_shape=jax.ShapeDtypeStruct((BUF_ROWS, HALF), jnp.int32),
    )(buf, Wk, Wv)


def _gate_body(r_ref, y_ref, out_ref):
    out_ref[...] = _unpack_rows(r_ref[...]) * _unpack_rows(y_ref[...])


def _gate(r, y):
    return pl.pallas_call(
        _gate_body,
        grid=(B,),
        in_specs=[
            pl.BlockSpec((T, HALF), lambda b: (b, 0)),
            pl.BlockSpec((T, HALF), lambda b: (b, 0)),
        ],
        out_specs=pl.BlockSpec((T, N_EMBD), lambda b: (b, 0)),
        out_shape=jax.ShapeDtypeStruct((N, N_EMBD), jnp.float32),
    )(r, y)


def kernel(x, shift_state, time_maa_k, time_maa_r, Wk, Wv, Wr, token_ids):
    maak2 = time_maa_k.reshape(1, N_EMBD)
    maar2 = time_maa_r.reshape(1, N_EMBD)
    ss3 = shift_state.reshape(B, 1, N_EMBD)
    tid3 = token_ids.reshape(B, 1, T)

    xk, r, loc2 = _mix_route(x, ss3, maak2, maar2, Wr, tid3)

    buf = _make_dispatch()(xk, loc2)
    outbuf = _ffn(buf, Wk, Wv)
    y = _make_combine()(outbuf, loc2)
    out = _gate(r, y)
    return out.reshape(B, T, N_EMBD), x[:, -1]
